# baseline (speedup 1.0000x reference)
import functools
import math

import jax
import jax.numpy as jnp
from jax import lax
from jax.experimental import pallas as pl
from jax.experimental.pallas import tpu as pltpu
from jax.experimental.pallas import tpu_sc as plsc

D_MODEL = 128
SCALE = math.sqrt(D_MODEL)

_NC = 2
_NS = 16
_NW = _NC * _NS
_LANES = 16

_CH = 128
_PAIR = 2
_ROWS = _CH * _PAIR
_NSLOT = 3


@functools.lru_cache(maxsize=None)
def _make_kernel(B: int):
    assert B % (_NW * _ROWS) == 0
    n_per_w = B // _NW
    n_pairs = n_per_w // _ROWS
    assert n_pairs > _NSLOT
    mesh = plsc.VectorSubcoreMesh(core_axis_name="c", subcore_axis_name="s")

    @functools.partial(
        pl.kernel,
        mesh=mesh,
        out_type=jax.ShapeDtypeStruct((B, D_MODEL), jnp.float32),
        scratch_types=[
            pltpu.VMEM((n_per_w,), jnp.int32),
            pltpu.VMEM((_NSLOT, _ROWS, D_MODEL), jnp.float32),
            pltpu.SemaphoreType.DMA((_NSLOT,)),
            pltpu.SemaphoreType.DMA((_NSLOT,)),
        ],
    )
    def gather_scale(x_hbm, table_hbm, out_hbm, idx_all, rows, gsem, osem):
        wid = lax.axis_index("s") * _NC + lax.axis_index("c")
        base = wid * n_per_w

        pltpu.sync_copy(x_hbm.at[pl.ds(base, n_per_w)], idx_all)

        def fire_gathers(q, s):
            for k in range(_PAIR):
                pltpu.async_copy(
                    table_hbm.at[idx_all.at[pl.ds(q * _ROWS + k * _CH, _CH)]],
                    rows.at[s].at[pl.ds(k * _CH, _CH)], gsem.at[s])

        def wait_gathers(q, s):
            for k in range(_PAIR):
                pltpu.make_async_copy(
                    table_hbm.at[idx_all.at[pl.ds(q * _ROWS + k * _CH, _CH)]],
                    rows.at[s].at[pl.ds(k * _CH, _CH)], gsem.at[s]).wait()

        def wait_out(s):
            pltpu.make_async_copy(
                rows.at[s], out_hbm.at[pl.ds(base, _ROWS)], osem.at[s]).wait()

        fire_gathers(0, 0)
        fire_gathers(1, 1)

        def pair_body(q, carry):
            s = lax.rem(q, _NSLOT)
            wait_gathers(q, s)

            @pl.when(q + _PAIR < n_pairs)
            def _prefetch():
                s2 = lax.rem(q + _PAIR, _NSLOT)

                @pl.when(q >= 1)
                def _wait_prev_out():
                    wait_out(s2)

                fire_gathers(q + _PAIR, s2)

            buf = rows.at[s]

            def row_body(i, c):
                for j in range(D_MODEL // _LANES):
                    sl = pl.ds(j * _LANES, _LANES)
                    buf[i, sl] = buf[i, sl] * SCALE
                return c

            lax.fori_loop(0, _ROWS, row_body, 0, unroll=False)
            pltpu.async_copy(buf, out_hbm.at[pl.ds(base + q * _ROWS, _ROWS)],
                             osem.at[s])
            return carry

        lax.fori_loop(0, n_pairs, pair_body, 0, unroll=False)

        for k in range(_NSLOT):
            wait_out((n_pairs - _NSLOT + k) % _NSLOT)

    return gather_scale


def kernel(x, table):
    S, T = x.shape
    B = S * T
    x_flat = x.reshape(B).astype(jnp.int32)
    out = _make_kernel(B)(x_flat, table)
    return out.reshape(S, T, D_MODEL)


# speedup vs baseline: 3.7748x; 3.7748x over previous
import functools
import math

import jax
import jax.numpy as jnp
from jax import lax
from jax.experimental import pallas as pl
from jax.experimental.pallas import tpu as pltpu
from jax.experimental.pallas import tpu_sc as plsc

D_MODEL = 128
SCALE = math.sqrt(D_MODEL)

_NC = 2
_NS = 16
_NW = _NC * _NS
_LANES = 16

_ROWS = 200
_SPLITS = ((0, 128), (128, 72))
_NBUF = 4
_A = 2


@functools.lru_cache(maxsize=None)
def _make_kernel(B: int):
    assert B % (_NW * _ROWS * _NBUF) == 0
    n_per_w = B // _NW
    n_blocks = n_per_w // _ROWS
    n_trips = n_blocks // _NBUF
    assert n_blocks >= _NBUF + _A
    mesh = plsc.VectorSubcoreMesh(core_axis_name="c", subcore_axis_name="s")

    @functools.partial(
        pl.kernel,
        mesh=mesh,
        out_type=jax.ShapeDtypeStruct((B, D_MODEL), jnp.float32),
        scratch_types=[
            pltpu.VMEM((n_per_w,), jnp.int32),
            pltpu.VMEM((_NBUF, _ROWS, D_MODEL), jnp.float32),
            pltpu.SemaphoreType.DMA((_NBUF,)),
            pltpu.SemaphoreType.DMA((_NBUF,)),
        ],
    )
    def gather_scale(x_hbm, table_hbm, out_hbm, idx_all, rows, gsem, osem):
        wid = lax.axis_index("s") * _NC + lax.axis_index("c")
        base = wid * n_per_w

        pltpu.sync_copy(x_hbm.at[pl.ds(base, n_per_w)], idx_all)

        def fire_gathers(g, b):
            for o, ln in _SPLITS:
                pltpu.async_copy(
                    table_hbm.at[idx_all.at[pl.ds(g * _ROWS + o, ln)]],
                    rows.at[b].at[pl.ds(o, ln)], gsem.at[b])

        def wait_gathers(g, b):
            for o, ln in _SPLITS:
                pltpu.make_async_copy(
                    table_hbm.at[idx_all.at[pl.ds(g * _ROWS + o, ln)]],
                    rows.at[b].at[pl.ds(o, ln)], gsem.at[b]).wait()

        def wait_out(b):
            pltpu.make_async_copy(
                rows.at[b], out_hbm.at[pl.ds(base, _ROWS)], osem.at[b]).wait()

        for b in range(_A):
            fire_gathers(b, b)

        def trip_body(t, carry):
            for b in range(_NBUF):
                g = t * _NBUF + b
                off = base + g * _ROWS
                wait_gathers(g, b)
                buf = rows.at[b]

                def row_body(i, c):
                    for j in range(D_MODEL // _LANES):
                        sl = pl.ds(j * _LANES, _LANES)
                        buf[i, sl] = buf[i, sl] * SCALE
                    return c

                lax.fori_loop(0, _ROWS, row_body, 0, unroll=False)
                pltpu.async_copy(buf, out_hbm.at[pl.ds(off, _ROWS)],
                                 osem.at[b])

                bq = (b + _A) % _NBUF
                if b + _A < _NBUF:
                    @pl.when(t > 0)
                    def _wait_prev_out():
                        wait_out(bq)
                    fire_gathers(g + _A, bq)
                else:
                    @pl.when(t < n_trips - 1)
                    def _prefetch_next_trip():
                        wait_out(bq)
                        fire_gathers(g + _A, bq)
            return carry

        lax.fori_loop(0, n_trips, trip_body, 0, unroll=False)

        for k in range(_A):
            wait_out((n_blocks - _A + k) % _NBUF)

    return gather_scale


def kernel(x, table):
    S, T = x.shape
    B = S * T
    x_flat = x.reshape(B).astype(jnp.int32)
    out = _make_kernel(B)(x_flat, table)
    return out.reshape(S, T, D_MODEL)


# speedup vs baseline: 3.8201x; 1.0120x over previous
import functools
import math

import jax
import jax.numpy as jnp
from jax import lax
from jax.experimental import pallas as pl
from jax.experimental.pallas import tpu as pltpu
from jax.experimental.pallas import tpu_sc as plsc

D_MODEL = 128
SCALE = math.sqrt(D_MODEL)

_NC = 2
_NS = 16
_NW = _NC * _NS
_LANES = 16

_CH = 128
_NBUF = 4
_A = 2


@functools.lru_cache(maxsize=None)
def _make_kernel(B: int):
    assert B % (_NW * _CH * _NBUF) == 0
    n_per_w = B // _NW
    n_chunks = n_per_w // _CH
    n_trips = n_chunks // _NBUF
    assert n_chunks >= _NBUF + _A
    mesh = plsc.VectorSubcoreMesh(core_axis_name="c", subcore_axis_name="s")

    @functools.partial(
        pl.kernel,
        mesh=mesh,
        out_type=jax.ShapeDtypeStruct((B, D_MODEL), jnp.float32),
        scratch_types=[
            pltpu.VMEM((n_per_w,), jnp.int32),
            pltpu.VMEM((_NBUF, _CH, D_MODEL), jnp.float32),
            pltpu.SemaphoreType.DMA((_NBUF,)),
            pltpu.SemaphoreType.DMA((_NBUF,)),
        ],
    )
    def gather_scale(x_hbm, table_hbm, out_hbm, idx_all, rows, gsem, osem):
        wid = lax.axis_index("s") * _NC + lax.axis_index("c")
        base = wid * n_per_w

        pltpu.sync_copy(x_hbm.at[pl.ds(base, n_per_w)], idx_all)

        def fire_gather(g, b):
            pltpu.async_copy(
                table_hbm.at[idx_all.at[pl.ds(g * _CH, _CH)]],
                rows.at[b], gsem.at[b])

        def wait_gather(g, b):
            pltpu.make_async_copy(
                table_hbm.at[idx_all.at[pl.ds(g * _CH, _CH)]],
                rows.at[b], gsem.at[b]).wait()

        def wait_out(b):
            pltpu.make_async_copy(
                rows.at[b], out_hbm.at[pl.ds(base, _CH)], osem.at[b]).wait()

        for b in range(_A):
            fire_gather(b, b)

        def trip_body(t, carry):
            for b in range(_NBUF):
                g = t * _NBUF + b
                off = base + g * _CH
                wait_gather(g, b)
                buf = rows.at[b]

                def row_body(i, c):
                    for j in range(D_MODEL // _LANES):
                        sl = pl.ds(j * _LANES, _LANES)
                        buf[i, sl] = buf[i, sl] * SCALE
                    return c

                lax.fori_loop(0, _CH, row_body, 0, unroll=False)
                pltpu.async_copy(buf, out_hbm.at[pl.ds(off, _CH)],
                                 osem.at[b])

                bq = (b + _A) % _NBUF
                if b + _A < _NBUF:
                    @pl.when(t > 0)
                    def _wait_prev_out():
                        wait_out(bq)
                    fire_gather(g + _A, bq)
                else:
                    @pl.when(t < n_trips - 1)
                    def _prefetch_next_trip():
                        wait_out(bq)
                        fire_gather(g + _A, bq)
            return carry

        lax.fori_loop(0, n_trips, trip_body, 0, unroll=False)

        for k in range(_A):
            wait_out((n_chunks - _A + k) % _NBUF)

    return gather_scale


def kernel(x, table):
    S, T = x.shape
    B = S * T
    x_flat = x.reshape(B).astype(jnp.int32)
    out = _make_kernel(B)(x_flat, table)
    return out.reshape(S, T, D_MODEL)
